# baseline (speedup 1.0000x reference)
import jax
import jax.numpy as jnp
from jax import lax
from jax.experimental import pallas as pl
from jax.experimental.pallas import tpu as pltpu
from jax.experimental.pallas import tpu_sc as plsc

EMBED_DIM = 32
LANES = 16
PAD_DIM = 128
CHUNK = 128
NUM_CORES = 2
NUM_SUBCORES = 16
NUM_WORKERS = NUM_CORES * NUM_SUBCORES


def kernel(input, table):
    batch, hist = input.shape
    num_indices = batch * hist
    b_per_w = num_indices // NUM_WORKERS
    nchunks = b_per_w // CHUNK
    assert nchunks % 2 == 0 and nchunks >= 4
    indices = input.reshape(num_indices)
    table_pad = jnp.pad(table, ((0, 0), (0, PAD_DIM - EMBED_DIM)))

    mesh = plsc.VectorSubcoreMesh(core_axis_name="core",
                                  subcore_axis_name="subcore")

    @pl.kernel(
        out_type=jax.ShapeDtypeStruct((num_indices * EMBED_DIM,),
                                      jnp.float32),
        mesh=mesh,
        scratch_types=[
            pltpu.VMEM((b_per_w,), jnp.int32),
            pltpu.VMEM((CHUNK, PAD_DIM), jnp.float32),
            pltpu.VMEM((CHUNK, PAD_DIM), jnp.float32),
            pltpu.VMEM((CHUNK * EMBED_DIM,), jnp.float32),
            pltpu.SemaphoreType.DMA,
            pltpu.SemaphoreType.DMA,
        ],
    )
    def gather_kernel(tab_hbm, idx_hbm, out_hbm, idx_v, rows0, rows1,
                      compact, sem0, sem1):
        wid = lax.axis_index("subcore") * NUM_CORES + lax.axis_index("core")
        base = wid * b_per_w
        pltpu.sync_copy(idx_hbm.at[pl.ds(base, b_per_w)], idx_v)

        def gather(chunk, rows, sem):
            pltpu.async_copy(
                tab_hbm.at[idx_v.at[pl.ds(chunk * CHUNK, CHUNK)]], rows, sem)

        def wait_gather(chunk, rows, sem):
            pltpu.make_async_copy(
                tab_hbm.at[idx_v.at[pl.ds(chunk * CHUNK, CHUNK)]], rows,
                sem).wait()

        def extract_and_write(chunk, rows):
            @pl.loop(0, CHUNK)
            def _(r):
                row = rows.at[r]
                compact[pl.ds(r * EMBED_DIM, LANES)] = row[pl.ds(0, LANES)]
                compact[pl.ds(r * EMBED_DIM + LANES, LANES)] = (
                    row[pl.ds(LANES, LANES)])

            pltpu.sync_copy(
                compact,
                out_hbm.at[pl.ds((base + chunk * CHUNK) * EMBED_DIM,
                                 CHUNK * EMBED_DIM)])

        gather(0, rows0, sem0)
        gather(1, rows1, sem1)

        @pl.loop(0, nchunks - 2, step=2)
        def _(k):
            wait_gather(k, rows0, sem0)
            extract_and_write(k, rows0)
            gather(k + 2, rows0, sem0)
            wait_gather(k + 1, rows1, sem1)
            extract_and_write(k + 1, rows1)
            gather(k + 3, rows1, sem1)

        wait_gather(nchunks - 2, rows0, sem0)
        extract_and_write(nchunks - 2, rows0)
        wait_gather(nchunks - 1, rows1, sem1)
        extract_and_write(nchunks - 1, rows1)

    out_flat = gather_kernel(table_pad, indices)
    return out_flat.reshape(batch, hist, EMBED_DIM)
